# baseline (speedup 1.0000x reference)
import functools

import jax
import jax.numpy as jnp
from jax import lax
from jax.experimental import pallas as pl
from jax.experimental.pallas import tpu as pltpu

B = 64
N = 512
NP = 513
T = 16
KEY = 32
HID = 32
FUNC = 256
IN = 1024
BBLK = 8
GRID = B // BBLK


def _fused_kernel(ar_ref, utm_ref, ent_ref, sel_ref, kw_ref, kb_ref,
                  fw_ref, fb_ref, f1w_ref, f1b_ref, f2w_ref, f2b_ref,
                  ew_ref, eb_ref, endw_ref, wih_ref, whh_ref, bih_ref,
                  bhh_ref, out_ref, key_s, r_s, fs_s, m_s, cb_s, w2_s,
                  gxb_s, x0_s, fe_s, h_s):
    i = pl.program_id(0)

    @pl.when(i == 0)
    def _prep():
        key_s[:, N:NP, :] = jnp.broadcast_to(endw_ref[...][None, :, :],
                                             (B, 1, KEY))
        m_s[...] = lax.dot_general(ew_ref[...], f1w_ref[...],
                                   (((0,), (1,)), ((), ())),
                                   preferred_element_type=jnp.float32)
        cb_s[...] = lax.dot_general(eb_ref[...].reshape(1, IN), f1w_ref[...],
                                    (((1,), (1,)), ((), ())),
                                    preferred_element_type=jnp.float32)
        w2_s[...] = lax.dot_general(wih_ref[...], f2w_ref[...],
                                    (((1,), (0,)), ((), ())),
                                    preferred_element_type=jnp.float32)
        gxb = lax.dot_general(f2b_ref[...].reshape(1, KEY), wih_ref[...],
                              (((1,), (1,)), ((), ())),
                              preferred_element_type=jnp.float32)
        gxb = gxb + (bih_ref[...] + bhh_ref[...])[None, :]
        gxb_s[...] = lax.transpose(gxb, (1, 0))
        x0_s[...] = lax.dot_general(ar_ref[...], f1w_ref[...],
                                    (((1,), (1,)), ((), ())),
                                    preferred_element_type=jnp.float32
                                    ) + f1b_ref[...][None, :]
        fe = lax.dot_general(utm_ref[...], fw_ref[...],
                             (((1,), (1,)), ((), ())),
                             preferred_element_type=jnp.float32)
        fe_s[...] = jnp.maximum(fe + fb_ref[...][None, :], 0.0)

    @pl.when(i < GRID)
    def _stream():
        ent2 = ent_ref[...].reshape(BBLK * N, -1)
        kblk = lax.dot_general(ent2, kw_ref[...], (((1,), (1,)), ((), ())),
                               preferred_element_type=jnp.float32)
        kblk3 = (kblk + kb_ref[...][None, :]).reshape(BBLK, N, KEY)
        key_s[pl.ds(i * BBLK, BBLK), 0:N, :] = kblk3

        selb = sel_ref[pl.ds(i * BBLK, BBLK), :]
        iota_n = lax.broadcasted_iota(jnp.int32, (BBLK, T, NP), 2)
        ohb = selb[:, :, None] == iota_n
        j_iota = lax.broadcasted_iota(jnp.int32, (BBLK, T, NP), 1)
        fs_s[pl.ds(i * BBLK, BBLK), :] = jnp.min(
            jnp.where(ohb, j_iota, T + 1), axis=1)
        ohbf = ohb[:, :, 0:N].astype(jnp.float32)
        eselb = lax.dot_general(ohbf, kblk3, (((2,), (1,)), ((0,), (0,))),
                                preferred_element_type=jnp.float32)
        eselb = eselb * (1.0 / N)

        p_blk = lax.dot_general(eselb.reshape(BBLK * T, KEY), m_s[...],
                                (((1,), (0,)), ((), ())),
                                preferred_element_type=jnp.float32) + cb_s[...]
        p_blk = p_blk.reshape(BBLK, T, FUNC)
        xb = x0_s[pl.ds(i * BBLK, BBLK), :]
        feb = fe_s[pl.ds(i * BBLK, BBLK), :]
        r_rows = [jnp.maximum(xb + feb, 0.0)]
        for t in range(T):
            xb = xb + p_blk[:, t, :].reshape(BBLK, FUNC)
            r_rows.append(jnp.maximum(xb + feb, 0.0))
        r_blk = jnp.concatenate(r_rows, axis=0).reshape(T + 1, BBLK, FUNC)
        r_s[:, pl.ds(i * BBLK, BBLK), :] = r_blk

    def _emit_half(h_all, half):
        b0 = half * (B // 2)
        hh = lax.slice(h_all, (0, b0, 0), (T + 1, b0 + B // 2, HID))
        kk = key_s[b0:b0 + B // 2, :, :]
        q = lax.dot_general(hh, kk, (((2,), (2,)), ((1,), (0,))),
                            preferred_element_type=jnp.float32)
        t_iota = lax.broadcasted_iota(jnp.int32, (B // 2, T + 1, NP), 1)
        fs = fs_s[b0:b0 + B // 2, :]
        pen = jnp.where(t_iota > fs[:, None, :], -1e9, 0.0)
        out_ref[...] = q + pen

    @pl.when(i == GRID - 1)
    def _decode():
        w2 = w2_s[...]
        gxb_col = gxb_s[...]
        gx_list = [
            lax.dot_general(w2, r_s[t], (((1,), (1,)), ((), ())),
                            preferred_element_type=jnp.float32) + gxb_col
            for t in range(T + 1)
        ]

        h_t = jnp.zeros((HID, B), dtype=jnp.float32)
        c_t = jnp.zeros((HID, B), dtype=jnp.float32)
        h_rows = []
        for t in range(T + 1):
            g = gx_list[t] + lax.dot_general(
                whh_ref[...], h_t, (((1,), (0,)), ((), ())),
                preferred_element_type=jnp.float32)
            gi = g[0:HID, :]
            gf = g[HID:2 * HID, :]
            gg = g[2 * HID:3 * HID, :]
            go = g[3 * HID:4 * HID, :]
            c_t = jax.nn.sigmoid(gf) * c_t + jax.nn.sigmoid(gi) * jnp.tanh(gg)
            h_t = jax.nn.sigmoid(go) * jnp.tanh(c_t)
            h_rows.append(lax.transpose(h_t, (1, 0)))

        h_all = jnp.concatenate(h_rows, axis=0).reshape(T + 1, B, HID)
        h_s[...] = h_all
        _emit_half(h_all, 0)

    @pl.when(i == GRID)
    def _decode_tail():
        _emit_half(h_s[...], 1)


@jax.jit
def _run(autoregressive_embedding, unit_type_mask, entity_embedding,
         selected_units, key_fc_w, key_fc_b, func_fc_w, func_fc_b,
         fc1_w, fc1_b, fc2_w, fc2_b, embed_fc_w, embed_fc_b,
         end_embedding, lstm_w_ih, lstm_w_hh, lstm_b_ih, lstm_b_hh):
    full = lambda a: pl.BlockSpec(a.shape, lambda i: (0,) * a.ndim)
    args = (autoregressive_embedding, unit_type_mask, entity_embedding,
            selected_units, key_fc_w, key_fc_b, func_fc_w, func_fc_b,
            fc1_w, fc1_b, fc2_w, fc2_b, embed_fc_w, embed_fc_b,
            end_embedding, lstm_w_ih, lstm_w_hh, lstm_b_ih, lstm_b_hh)
    in_specs = [full(a) for a in args]
    in_specs[2] = pl.BlockSpec((BBLK, N, entity_embedding.shape[2]),
                               lambda i: (jnp.minimum(i, GRID - 1), 0, 0))
    return pl.pallas_call(
        _fused_kernel,
        grid=(GRID + 1,),
        in_specs=in_specs,
        out_specs=pl.BlockSpec((B // 2, T + 1, NP),
                               lambda i: (jnp.maximum(i - (GRID - 1), 0), 0, 0)),
        out_shape=jax.ShapeDtypeStruct((B, T + 1, NP), jnp.float32),
        scratch_shapes=[
            pltpu.VMEM((B, NP, KEY), jnp.float32),
            pltpu.VMEM((T + 1, B, FUNC), jnp.float32),
            pltpu.VMEM((B, NP), jnp.int32),
            pltpu.VMEM((KEY, FUNC), jnp.float32),
            pltpu.VMEM((1, FUNC), jnp.float32),
            pltpu.VMEM((4 * HID, FUNC), jnp.float32),
            pltpu.VMEM((4 * HID, 1), jnp.float32),
            pltpu.VMEM((B, FUNC), jnp.float32),
            pltpu.VMEM((B, FUNC), jnp.float32),
            pltpu.VMEM((T + 1, B, HID), jnp.float32),
        ],
    )(*args)


def kernel(autoregressive_embedding, unit_type_mask, entity_embedding,
           entity_mask, selected_units, key_fc_w, key_fc_b, func_fc_w,
           func_fc_b, fc1_w, fc1_b, fc2_w, fc2_b, embed_fc_w, embed_fc_b,
           end_embedding, lstm_w_ih, lstm_w_hh, lstm_b_ih, lstm_b_hh):
    return _run(autoregressive_embedding, unit_type_mask, entity_embedding,
                selected_units, key_fc_w, key_fc_b, func_fc_w, func_fc_b,
                fc1_w, fc1_b, fc2_w, fc2_b, embed_fc_w, embed_fc_b,
                end_embedding, lstm_w_ih, lstm_w_hh, lstm_b_ih, lstm_b_hh)


# speedup vs baseline: 1.0036x; 1.0036x over previous
import functools

import jax
import jax.numpy as jnp
from jax import lax
from jax.experimental import pallas as pl
from jax.experimental.pallas import tpu as pltpu

B = 64
N = 512
NP = 513
T = 16
KEY = 32
HID = 32
FUNC = 256
IN = 1024
BBLK = 8
GRID = B // BBLK


def _fused_kernel(ar_ref, utm_ref, ent_ref, sel_ref, kw_ref, kb_ref,
                  fw_ref, fb_ref, f1w_ref, f1b_ref, f2w_ref, f2b_ref,
                  ew_ref, eb_ref, endw_ref, wih_ref, whh_ref, bih_ref,
                  bhh_ref, out_ref, key_s, esel_s, fs_s, m_s, cb_s, w2_s,
                  gxb_s, x0_s, fe_s, h_s):
    i = pl.program_id(0)

    @pl.when(i < GRID)
    def _stream():
        ent2 = ent_ref[...].reshape(BBLK * N, -1)
        kblk = lax.dot_general(ent2, kw_ref[...], (((1,), (1,)), ((), ())),
                               preferred_element_type=jnp.float32)
        kblk3 = (kblk + kb_ref[...][None, :]).reshape(BBLK, N, KEY)
        key_s[pl.ds(i * BBLK, BBLK), 0:N, :] = kblk3

        selb = sel_ref[pl.ds(i * BBLK, BBLK), :]
        iota_n = lax.broadcasted_iota(jnp.int32, (BBLK, T, NP), 2)
        ohb = selb[:, :, None] == iota_n
        j_iota = lax.broadcasted_iota(jnp.int32, (BBLK, T, NP), 1)
        fs_s[pl.ds(i * BBLK, BBLK), :] = jnp.min(
            jnp.where(ohb, j_iota, T + 1), axis=1)
        ohbf = ohb[:, :, 0:N].astype(jnp.float32)
        eselb = lax.dot_general(ohbf, kblk3, (((2,), (1,)), ((0,), (0,))),
                                preferred_element_type=jnp.float32)
        esel_s[pl.ds(i * BBLK, BBLK), :, :] = eselb * (1.0 / N)

    @pl.when(i == 0)
    def _prep():
        key_s[:, N:NP, :] = jnp.broadcast_to(endw_ref[...][None, :, :],
                                             (B, 1, KEY))
        m_s[...] = lax.dot_general(ew_ref[...], f1w_ref[...],
                                   (((0,), (1,)), ((), ())),
                                   preferred_element_type=jnp.float32)
        cb_s[...] = lax.dot_general(eb_ref[...].reshape(1, IN), f1w_ref[...],
                                    (((1,), (1,)), ((), ())),
                                    preferred_element_type=jnp.float32)
        w2_s[...] = lax.dot_general(wih_ref[...], f2w_ref[...],
                                    (((1,), (0,)), ((), ())),
                                    preferred_element_type=jnp.float32)
        gxb = lax.dot_general(f2b_ref[...].reshape(1, KEY), wih_ref[...],
                              (((1,), (1,)), ((), ())),
                              preferred_element_type=jnp.float32)
        gxb = gxb + (bih_ref[...] + bhh_ref[...])[None, :]
        gxb_s[...] = lax.transpose(gxb, (1, 0))
        x0_s[...] = lax.dot_general(ar_ref[...], f1w_ref[...],
                                    (((1,), (1,)), ((), ())),
                                    preferred_element_type=jnp.float32
                                    ) + f1b_ref[...][None, :]
        fe = lax.dot_general(utm_ref[...], fw_ref[...],
                             (((1,), (1,)), ((), ())),
                             preferred_element_type=jnp.float32)
        fe_s[...] = jnp.maximum(fe + fb_ref[...][None, :], 0.0)

    def _emit_half(h_all, half):
        b0 = half * (B // 2)
        hh = lax.slice(h_all, (0, b0, 0), (T + 1, b0 + B // 2, HID))
        kk = key_s[b0:b0 + B // 2, :, :]
        q = lax.dot_general(hh, kk, (((2,), (2,)), ((1,), (0,))),
                            preferred_element_type=jnp.float32)
        t_iota = lax.broadcasted_iota(jnp.int32, (B // 2, T + 1, NP), 1)
        fs = fs_s[b0:b0 + B // 2, :]
        pen = jnp.where(t_iota > fs[:, None, :], -1e9, 0.0)
        out_ref[...] = q + pen

    @pl.when(i == GRID - 1)
    def _decode():
        p_all = lax.dot_general(esel_s[...].reshape(B * T, KEY), m_s[...],
                                (((1,), (0,)), ((), ())),
                                preferred_element_type=jnp.float32) + cb_s[...]
        p_all = p_all.reshape(B, T, FUNC)

        fe = fe_s[...]
        xi = x0_s[...]
        r_rows = [jnp.maximum(xi + fe, 0.0)]
        for t in range(T):
            xi = xi + p_all[:, t, :].reshape(B, FUNC)
            r_rows.append(jnp.maximum(xi + fe, 0.0))

        w2 = w2_s[...]
        gxb_col = gxb_s[...]
        gx_list = [
            lax.dot_general(w2, r, (((1,), (1,)), ((), ())),
                            preferred_element_type=jnp.float32) + gxb_col
            for r in r_rows
        ]

        h_t = jnp.zeros((HID, B), dtype=jnp.float32)
        c_t = jnp.zeros((HID, B), dtype=jnp.float32)
        h_rows = []
        for t in range(T + 1):
            g = gx_list[t] + lax.dot_general(
                whh_ref[...], h_t, (((1,), (0,)), ((), ())),
                preferred_element_type=jnp.float32)
            gi = g[0:HID, :]
            gf = g[HID:2 * HID, :]
            gg = g[2 * HID:3 * HID, :]
            go = g[3 * HID:4 * HID, :]
            c_t = jax.nn.sigmoid(gf) * c_t + jax.nn.sigmoid(gi) * jnp.tanh(gg)
            h_t = jax.nn.sigmoid(go) * jnp.tanh(c_t)
            h_rows.append(lax.transpose(h_t, (1, 0)))

        h_all = jnp.concatenate(h_rows, axis=0).reshape(T + 1, B, HID)
        h_s[...] = h_all
        _emit_half(h_all, 0)

    @pl.when(i == GRID)
    def _decode_tail():
        _emit_half(h_s[...], 1)


@jax.jit
def _run(autoregressive_embedding, unit_type_mask, entity_embedding,
         selected_units, key_fc_w, key_fc_b, func_fc_w, func_fc_b,
         fc1_w, fc1_b, fc2_w, fc2_b, embed_fc_w, embed_fc_b,
         end_embedding, lstm_w_ih, lstm_w_hh, lstm_b_ih, lstm_b_hh):
    full = lambda a: pl.BlockSpec(a.shape, lambda i: (0,) * a.ndim)
    args = (autoregressive_embedding, unit_type_mask, entity_embedding,
            selected_units, key_fc_w, key_fc_b, func_fc_w, func_fc_b,
            fc1_w, fc1_b, fc2_w, fc2_b, embed_fc_w, embed_fc_b,
            end_embedding, lstm_w_ih, lstm_w_hh, lstm_b_ih, lstm_b_hh)
    in_specs = [full(a) for a in args]
    in_specs[2] = pl.BlockSpec((BBLK, N, entity_embedding.shape[2]),
                               lambda i: (jnp.minimum(i, GRID - 1), 0, 0))
    return pl.pallas_call(
        _fused_kernel,
        grid=(GRID + 1,),
        in_specs=in_specs,
        out_specs=pl.BlockSpec((B // 2, T + 1, NP),
                               lambda i: (jnp.maximum(i - (GRID - 1), 0), 0, 0)),
        out_shape=jax.ShapeDtypeStruct((B, T + 1, NP), jnp.float32),
        scratch_shapes=[
            pltpu.VMEM((B, NP, KEY), jnp.float32),
            pltpu.VMEM((B, T, KEY), jnp.float32),
            pltpu.VMEM((B, NP), jnp.int32),
            pltpu.VMEM((KEY, FUNC), jnp.float32),
            pltpu.VMEM((1, FUNC), jnp.float32),
            pltpu.VMEM((4 * HID, FUNC), jnp.float32),
            pltpu.VMEM((4 * HID, 1), jnp.float32),
            pltpu.VMEM((B, FUNC), jnp.float32),
            pltpu.VMEM((B, FUNC), jnp.float32),
            pltpu.VMEM((T + 1, B, HID), jnp.float32),
        ],
    )(*args)


def kernel(autoregressive_embedding, unit_type_mask, entity_embedding,
           entity_mask, selected_units, key_fc_w, key_fc_b, func_fc_w,
           func_fc_b, fc1_w, fc1_b, fc2_w, fc2_b, embed_fc_w, embed_fc_b,
           end_embedding, lstm_w_ih, lstm_w_hh, lstm_b_ih, lstm_b_hh):
    return _run(autoregressive_embedding, unit_type_mask, entity_embedding,
                selected_units, key_fc_w, key_fc_b, func_fc_w, func_fc_b,
                fc1_w, fc1_b, fc2_w, fc2_b, embed_fc_w, embed_fc_b,
                end_embedding, lstm_w_ih, lstm_w_hh, lstm_b_ih, lstm_b_hh)
